# baseline (speedup 1.0000x reference)
import functools

import jax
import jax.numpy as jnp
import numpy as np
from jax.experimental import pallas as pl
from jax.experimental.pallas import tpu as pltpu


def _np_threefry2x32(k1, k2, c0, c1):
    ks = (int(k1), int(k2), int(k1) ^ int(k2) ^ 0x1BD11BDA)
    x0 = (int(c0) + ks[0]) & 0xFFFFFFFF
    x1 = (int(c1) + ks[1]) & 0xFFFFFFFF
    rots = ((13, 15, 26, 6), (17, 29, 16, 24))
    inj = ((ks[1], ks[2]), (ks[2], ks[0]), (ks[0], ks[1]),
           (ks[1], ks[2]), (ks[2], ks[0]))
    for i in range(5):
        for r in rots[i % 2]:
            x0 = (x0 + x1) & 0xFFFFFFFF
            x1 = ((x1 << r) | (x1 >> (32 - r))) & 0xFFFFFFFF
            x1 = x0 ^ x1
        a, b = inj[i]
        x0 = (x0 + a) & 0xFFFFFFFF
        x1 = (x1 + b + i + 1) & 0xFFFFFFFF
    return x0, x1


_MASK_K1, _MASK_K2 = _np_threefry2x32(0, 0, 0, 1)

_Q_C = (0.0003539162583878881, 0.0026938206958895595,
        -0.0913519034237298, 0.5014342202091804)
_L_CLAMP = -5.35


_KS = (_MASK_K1, _MASK_K2, _MASK_K1 ^ _MASK_K2 ^ 0x1BD11BDA)
_INJ = tuple(
    (_KS[(i + 1) % 3], (_KS[(i + 2) % 3] + i + 1) & 0xFFFFFFFF)
    for i in range(5)
)


def _dropout_block_kernel(x_ref, iota_ref, o_ref, *, block_elems):
    pid = pl.program_id(0)

    base = pid.astype(jnp.uint32) * jnp.uint32(block_elems) + jnp.uint32(_KS[1])
    x0 = jnp.full(x_ref.shape, jnp.uint32(_KS[0]), dtype=jnp.uint32)
    x1 = iota_ref[...] + base

    rots = ((13, 15, 26, 6), (17, 29, 16, 24))
    for i in range(5):
        for r in rots[i % 2]:
            x0 = x0 + x1
            x1 = (x1 << r) | (x1 >> (32 - r))
            x1 = x0 ^ x1
        a, b = _INJ[i]
        x0 = x0 + jnp.uint32(a)
        x1 = x1 + jnp.uint32(b)

    bits = x0 ^ x1

    u = pltpu.bitcast((bits >> 9) | jnp.uint32(0x40000000), jnp.float32)
    u = u - jnp.float32(3.0)

    om = jnp.float32(1.0) - u * u
    ell = jnp.maximum(jnp.log2(om), jnp.float32(_L_CLAMP))
    q = jnp.float32(_Q_C[0])
    for c in _Q_C[1:]:
        q = jnp.float32(c) + q * ell
    m2 = jnp.clip(jnp.float32(1.0) + q * u, 0.0, 2.0)
    o_ref[...] = m2 * x_ref[...]


@jax.jit
def kernel(x):
    b, s, d = x.shape
    nrows = b * s
    ncols = d

    block_rows = 128
    grid = nrows // block_rows
    block_elems = block_rows * ncols

    iota = jnp.arange(block_elems, dtype=jnp.uint32).reshape(block_rows, ncols)

    out = pl.pallas_call(
        functools.partial(_dropout_block_kernel, block_elems=block_elems),
        grid=(grid,),
        in_specs=[
            pl.BlockSpec((block_rows, ncols), lambda i: (i, 0)),
            pl.BlockSpec((block_rows, ncols), lambda i: (0, 0)),
        ],
        out_specs=pl.BlockSpec((block_rows, ncols), lambda i: (i, 0)),
        out_shape=jax.ShapeDtypeStruct((nrows, ncols), x.dtype),
    )(x.reshape(nrows, ncols), iota)
    return out.reshape(b, s, d)


# speedup vs baseline: 1.0012x; 1.0012x over previous
import functools

import jax
import jax.numpy as jnp
import numpy as np
from jax.experimental import pallas as pl
from jax.experimental.pallas import tpu as pltpu


def _np_threefry2x32(k1, k2, c0, c1):
    ks = (int(k1), int(k2), int(k1) ^ int(k2) ^ 0x1BD11BDA)
    x0 = (int(c0) + ks[0]) & 0xFFFFFFFF
    x1 = (int(c1) + ks[1]) & 0xFFFFFFFF
    rots = ((13, 15, 26, 6), (17, 29, 16, 24))
    inj = ((ks[1], ks[2]), (ks[2], ks[0]), (ks[0], ks[1]),
           (ks[1], ks[2]), (ks[2], ks[0]))
    for i in range(5):
        for r in rots[i % 2]:
            x0 = (x0 + x1) & 0xFFFFFFFF
            x1 = ((x1 << r) | (x1 >> (32 - r))) & 0xFFFFFFFF
            x1 = x0 ^ x1
        a, b = inj[i]
        x0 = (x0 + a) & 0xFFFFFFFF
        x1 = (x1 + b + i + 1) & 0xFFFFFFFF
    return x0, x1


_MASK_K1, _MASK_K2 = _np_threefry2x32(0, 0, 0, 1)

_Q_C = (0.0003539162583878881, 0.0026938206958895595,
        -0.0913519034237298, 0.5014342202091804)
_L_CLAMP = -5.35


_KS = (_MASK_K1, _MASK_K2, _MASK_K1 ^ _MASK_K2 ^ 0x1BD11BDA)
_INJ = tuple(
    (_KS[(i + 1) % 3], (_KS[(i + 2) % 3] + i + 1) & 0xFFFFFFFF)
    for i in range(5)
)


def _dropout_block_kernel(x_ref, iota_ref, o_ref, *, block_elems):
    pid = pl.program_id(0)

    base = pid.astype(jnp.uint32) * jnp.uint32(block_elems) + jnp.uint32(_KS[1])
    x0 = jnp.full(x_ref.shape, jnp.uint32(_KS[0]), dtype=jnp.uint32)
    x1 = iota_ref[...] + base

    rots = ((13, 15, 26, 6), (17, 29, 16, 24))
    for i in range(5):
        for r in rots[i % 2]:
            x0 = x0 + x1
            x1 = (x1 << r) | (x1 >> (32 - r))
            x1 = x0 ^ x1
        a, b = _INJ[i]
        x0 = x0 + jnp.uint32(a)
        x1 = x1 + jnp.uint32(b)

    bits = x0 ^ x1

    u = pltpu.bitcast((bits >> 9) | jnp.uint32(0x40000000), jnp.float32)
    u = u - jnp.float32(3.0)

    om = jnp.float32(1.0) - u * u
    ell = jnp.maximum(jnp.log2(om), jnp.float32(_L_CLAMP))
    q = jnp.float32(_Q_C[0])
    for c in _Q_C[1:]:
        q = jnp.float32(c) + q * ell
    m2 = jnp.clip(jnp.float32(1.0) + q * u, 0.0, 2.0)
    o_ref[...] = m2 * x_ref[...]


@jax.jit
def kernel(x):
    b, s, d = x.shape
    nrows = b * s
    ncols = d

    block_rows = 256
    grid = nrows // block_rows
    block_elems = block_rows * ncols

    iota = jnp.arange(block_elems, dtype=jnp.uint32).reshape(block_rows, ncols)

    out = pl.pallas_call(
        functools.partial(_dropout_block_kernel, block_elems=block_elems),
        grid=(grid,),
        in_specs=[
            pl.BlockSpec((block_rows, ncols), lambda i: (i, 0)),
            pl.BlockSpec((block_rows, ncols), lambda i: (0, 0)),
        ],
        out_specs=pl.BlockSpec((block_rows, ncols), lambda i: (i, 0)),
        out_shape=jax.ShapeDtypeStruct((nrows, ncols), x.dtype),
    )(x.reshape(nrows, ncols), iota)
    return out.reshape(b, s, d)


# speedup vs baseline: 1.0163x; 1.0151x over previous
import functools

import jax
import jax.numpy as jnp
import numpy as np
from jax.experimental import pallas as pl
from jax.experimental.pallas import tpu as pltpu


def _np_threefry2x32(k1, k2, c0, c1):
    ks = (int(k1), int(k2), int(k1) ^ int(k2) ^ 0x1BD11BDA)
    x0 = (int(c0) + ks[0]) & 0xFFFFFFFF
    x1 = (int(c1) + ks[1]) & 0xFFFFFFFF
    rots = ((13, 15, 26, 6), (17, 29, 16, 24))
    inj = ((ks[1], ks[2]), (ks[2], ks[0]), (ks[0], ks[1]),
           (ks[1], ks[2]), (ks[2], ks[0]))
    for i in range(5):
        for r in rots[i % 2]:
            x0 = (x0 + x1) & 0xFFFFFFFF
            x1 = ((x1 << r) | (x1 >> (32 - r))) & 0xFFFFFFFF
            x1 = x0 ^ x1
        a, b = inj[i]
        x0 = (x0 + a) & 0xFFFFFFFF
        x1 = (x1 + b + i + 1) & 0xFFFFFFFF
    return x0, x1


_MASK_K1, _MASK_K2 = _np_threefry2x32(0, 0, 0, 1)

_Q_C = (-0.00019475560862928183, -0.09774012094277562, 0.4983101550477552)
_L_CLAMP = -5.35


_KS = (_MASK_K1, _MASK_K2, _MASK_K1 ^ _MASK_K2 ^ 0x1BD11BDA)
_INJ = tuple(
    (_KS[(i + 1) % 3], (_KS[(i + 2) % 3] + i + 1) & 0xFFFFFFFF)
    for i in range(5)
)


def _dropout_block_kernel(x_ref, iota_ref, o_ref, *, block_elems):
    pid = pl.program_id(0)

    base = pid.astype(jnp.uint32) * jnp.uint32(block_elems) + jnp.uint32(_KS[1])
    x0 = jnp.full(x_ref.shape, jnp.uint32(_KS[0]), dtype=jnp.uint32)
    x1 = iota_ref[...] + base

    rots = ((13, 15, 26, 6), (17, 29, 16, 24))
    for i in range(5):
        for r in rots[i % 2]:
            x0 = x0 + x1
            x1 = (x1 << r) | (x1 >> (32 - r))
            x1 = x0 ^ x1
        a, b = _INJ[i]
        x0 = x0 + jnp.uint32(a)
        x1 = x1 + jnp.uint32(b)

    bits = x0 ^ x1

    u = pltpu.bitcast((bits >> 9) | jnp.uint32(0x40000000), jnp.float32)
    u = u - jnp.float32(3.0)

    om = jnp.float32(1.0) - u * u
    ell = jnp.maximum(jnp.log2(om), jnp.float32(_L_CLAMP))
    q = jnp.float32(_Q_C[0])
    for c in _Q_C[1:]:
        q = jnp.float32(c) + q * ell
    m2 = jnp.clip(jnp.float32(1.0) + q * u, 0.0, 2.0)
    o_ref[...] = m2 * x_ref[...]


@jax.jit
def kernel(x):
    b, s, d = x.shape
    nrows = b * s
    ncols = d

    block_rows = 256
    grid = nrows // block_rows
    block_elems = block_rows * ncols

    iota = jnp.arange(block_elems, dtype=jnp.uint32).reshape(block_rows, ncols)

    out = pl.pallas_call(
        functools.partial(_dropout_block_kernel, block_elems=block_elems),
        grid=(grid,),
        in_specs=[
            pl.BlockSpec((block_rows, ncols), lambda i: (i, 0)),
            pl.BlockSpec((block_rows, ncols), lambda i: (0, 0)),
        ],
        out_specs=pl.BlockSpec((block_rows, ncols), lambda i: (i, 0)),
        out_shape=jax.ShapeDtypeStruct((nrows, ncols), x.dtype),
    )(x.reshape(nrows, ncols), iota)
    return out.reshape(b, s, d)


# speedup vs baseline: 1.0163x; 1.0000x over previous
import functools

import jax
import jax.numpy as jnp
import numpy as np
from jax.experimental import pallas as pl
from jax.experimental.pallas import tpu as pltpu


def _np_threefry2x32(k1, k2, c0, c1):
    ks = (int(k1), int(k2), int(k1) ^ int(k2) ^ 0x1BD11BDA)
    x0 = (int(c0) + ks[0]) & 0xFFFFFFFF
    x1 = (int(c1) + ks[1]) & 0xFFFFFFFF
    rots = ((13, 15, 26, 6), (17, 29, 16, 24))
    inj = ((ks[1], ks[2]), (ks[2], ks[0]), (ks[0], ks[1]),
           (ks[1], ks[2]), (ks[2], ks[0]))
    for i in range(5):
        for r in rots[i % 2]:
            x0 = (x0 + x1) & 0xFFFFFFFF
            x1 = ((x1 << r) | (x1 >> (32 - r))) & 0xFFFFFFFF
            x1 = x0 ^ x1
        a, b = inj[i]
        x0 = (x0 + a) & 0xFFFFFFFF
        x1 = (x1 + b + i + 1) & 0xFFFFFFFF
    return x0, x1


_MASK_K1, _MASK_K2 = _np_threefry2x32(0, 0, 0, 1)

_Q_C = (-0.00019475560862928183, -0.09774012094277562, 0.4983101550477552)
_L_CLAMP = -5.35


_KS = (_MASK_K1, _MASK_K2, _MASK_K1 ^ _MASK_K2 ^ 0x1BD11BDA)
_INJ = tuple(
    (_KS[(i + 1) % 3], (_KS[(i + 2) % 3] + i + 1) & 0xFFFFFFFF)
    for i in range(5)
)


def _dropout_block_kernel(x_ref, iota_ref, o_ref, *, block_elems):
    pid = pl.program_id(0)

    base = pid.astype(jnp.uint32) * jnp.uint32(block_elems) + jnp.uint32(_KS[1])
    x0 = jnp.full(x_ref.shape, jnp.uint32(_KS[0]), dtype=jnp.uint32)
    x1 = iota_ref[...] + base

    rots = ((13, 15, 26, 6), (17, 29, 16, 24))
    for i in range(5):
        for r in rots[i % 2]:
            x0 = x0 + x1
            x1 = (x1 << r) | (x1 >> (32 - r))
            x1 = x0 ^ x1
        a, b = _INJ[i]
        x0 = x0 + jnp.uint32(a)
        x1 = x1 + jnp.uint32(b)

    bits = x0 ^ x1

    u = pltpu.bitcast((bits >> 9) | jnp.uint32(0x40000000), jnp.float32)
    u = u - jnp.float32(3.0)

    om = jnp.float32(1.0) - u * u
    ell = jnp.maximum(jnp.log2(om), jnp.float32(_L_CLAMP))
    q = jnp.float32(_Q_C[0])
    for c in _Q_C[1:]:
        q = jnp.float32(c) + q * ell
    m2 = jnp.clip(jnp.float32(1.0) + q * u, 0.0, 2.0)
    o_ref[...] = m2 * x_ref[...]


@jax.jit
def kernel(x):
    b, s, d = x.shape
    nrows = b * s
    ncols = d

    block_rows = 256
    grid = nrows // block_rows
    block_elems = block_rows * ncols

    iota = jnp.arange(block_elems, dtype=jnp.uint32).reshape(block_rows, ncols)

    out = pl.pallas_call(
        functools.partial(_dropout_block_kernel, block_elems=block_elems),
        grid=(grid,),
        in_specs=[
            pl.BlockSpec((block_rows, ncols), lambda i: (i, 0)),
            pl.BlockSpec((block_rows, ncols), lambda i: (0, 0)),
        ],
        out_specs=pl.BlockSpec((block_rows, ncols), lambda i: (i, 0)),
        out_shape=jax.ShapeDtypeStruct((nrows, ncols), x.dtype),
        compiler_params=pltpu.CompilerParams(
            dimension_semantics=("parallel",),
        ),
    )(x.reshape(nrows, ncols), iota)
    return out.reshape(b, s, d)


# speedup vs baseline: 1.0190x; 1.0027x over previous
import functools

import jax
import jax.numpy as jnp
import numpy as np
from jax.experimental import pallas as pl
from jax.experimental.pallas import tpu as pltpu


def _np_threefry2x32(k1, k2, c0, c1):
    ks = (int(k1), int(k2), int(k1) ^ int(k2) ^ 0x1BD11BDA)
    x0 = (int(c0) + ks[0]) & 0xFFFFFFFF
    x1 = (int(c1) + ks[1]) & 0xFFFFFFFF
    rots = ((13, 15, 26, 6), (17, 29, 16, 24))
    inj = ((ks[1], ks[2]), (ks[2], ks[0]), (ks[0], ks[1]),
           (ks[1], ks[2]), (ks[2], ks[0]))
    for i in range(5):
        for r in rots[i % 2]:
            x0 = (x0 + x1) & 0xFFFFFFFF
            x1 = ((x1 << r) | (x1 >> (32 - r))) & 0xFFFFFFFF
            x1 = x0 ^ x1
        a, b = inj[i]
        x0 = (x0 + a) & 0xFFFFFFFF
        x1 = (x1 + b + i + 1) & 0xFFFFFFFF
    return x0, x1


_MASK_K1, _MASK_K2 = _np_threefry2x32(0, 0, 0, 1)

_Q_C = (-0.00019475560862928183, -0.09774012094277562, 0.4983101550477552)
_L_CLAMP = -5.35


_KS = (_MASK_K1, _MASK_K2, _MASK_K1 ^ _MASK_K2 ^ 0x1BD11BDA)
_INJ = tuple(
    (_KS[(i + 1) % 3], (_KS[(i + 2) % 3] + i + 1) & 0xFFFFFFFF)
    for i in range(5)
)


def _dropout_block_kernel(x_ref, o_ref, iota_scr, *, block_elems):
    pid = pl.program_id(0)

    @pl.when(pid == 0)
    def _init_iota():
        shape = iota_scr.shape
        r = jax.lax.broadcasted_iota(jnp.uint32, shape, 0)
        c = jax.lax.broadcasted_iota(jnp.uint32, shape, 1)
        iota_scr[...] = r * jnp.uint32(shape[1]) + c

    base = pid.astype(jnp.uint32) * jnp.uint32(block_elems) + jnp.uint32(_KS[1])
    x0 = jnp.full(x_ref.shape, jnp.uint32(_KS[0]), dtype=jnp.uint32)
    x1 = iota_scr[...] + base

    rots = ((13, 15, 26, 6), (17, 29, 16, 24))
    for i in range(5):
        for r in rots[i % 2]:
            x0 = x0 + x1
            x1 = (x1 << r) | (x1 >> (32 - r))
            x1 = x0 ^ x1
        a, b = _INJ[i]
        x0 = x0 + jnp.uint32(a)
        x1 = x1 + jnp.uint32(b)

    bits = x0 ^ x1

    u = pltpu.bitcast((bits >> 9) | jnp.uint32(0x40000000), jnp.float32)
    u = u - jnp.float32(3.0)

    om = jnp.float32(1.0) - u * u
    ell = jnp.maximum(jnp.log2(om), jnp.float32(_L_CLAMP))
    q = jnp.float32(_Q_C[0])
    for c in _Q_C[1:]:
        q = jnp.float32(c) + q * ell
    m2 = jnp.clip(jnp.float32(1.0) + q * u, 0.0, 2.0)
    o_ref[...] = m2 * x_ref[...]


@jax.jit
def kernel(x):
    b, s, d = x.shape
    nrows = b * s
    ncols = d

    block_rows = 256
    grid = nrows // block_rows
    block_elems = block_rows * ncols

    out = pl.pallas_call(
        functools.partial(_dropout_block_kernel, block_elems=block_elems),
        grid=(grid,),
        in_specs=[
            pl.BlockSpec((block_rows, ncols), lambda i: (i, 0)),
        ],
        out_specs=pl.BlockSpec((block_rows, ncols), lambda i: (i, 0)),
        out_shape=jax.ShapeDtypeStruct((nrows, ncols), x.dtype),
        scratch_shapes=[pltpu.VMEM((block_rows, ncols), jnp.uint32)],
    )(x.reshape(nrows, ncols))
    return out.reshape(b, s, d)
